# Initial kernel scaffold; baseline (speedup 1.0000x reference)
import jax
import jax.numpy as jnp
from jax.experimental import pallas as pl


def kernel(env, root_state, emb, W_ih, W_hh, b_ih, b_hh, W_out, b_out, beams, temp):
    raise NotImplementedError("write your pallas kernel here")


# speedup vs baseline: 4.9376x; 4.9376x over previous
import functools

import jax
import jax.numpy as jnp
from jax.experimental import pallas as pl
from jax.experimental.pallas import tpu as pltpu

HIDDEN = 1024
DECISIONS = 1024
ENV_DEPTH = 8
BATCH = 16
K = 64

PALLAS_LOGITS = True
PALLAS_GRU = True
PALLAS_TOPK = True


def _logits_body(s_ref, w_ref, b_ref, o_ref):
    o_ref[...] = jnp.dot(s_ref[...], w_ref[...],
                         preferred_element_type=jnp.float32) + b_ref[...]


def _pallas_logits(states, W_out, b_out2d):
    nb = states.shape[0]
    return pl.pallas_call(
        _logits_body,
        out_shape=jax.ShapeDtypeStruct((nb, DECISIONS), jnp.float32),
    )(states, W_out, b_out2d)


def _gru_body(x_ref, h_ref, wih_ref, whh_ref, bih_ref, bhh_ref, o_ref):
    x = x_ref[...]
    h = h_ref[...]
    dn = (((1,), (1,)), ((), ()))
    gi = jax.lax.dot_general(x, wih_ref[...], dn,
                             preferred_element_type=jnp.float32) + bih_ref[...]
    gh = jax.lax.dot_general(h, whh_ref[...], dn,
                             preferred_element_type=jnp.float32) + bhh_ref[...]
    H = HIDDEN
    i_r, i_z, i_n = gi[:, 0:H], gi[:, H:2 * H], gi[:, 2 * H:3 * H]
    h_r, h_z, h_n = gh[:, 0:H], gh[:, H:2 * H], gh[:, 2 * H:3 * H]
    r = jax.nn.sigmoid(i_r + h_r)
    z = jax.nn.sigmoid(i_z + h_z)
    n = jnp.tanh(i_n + r * h_n)
    o_ref[...] = (1.0 - z) * n + z * h


def _pallas_gru(x, h, W_ih, W_hh, b_ih2d, b_hh2d):
    nb = x.shape[0]
    return pl.pallas_call(
        _gru_body,
        out_shape=jax.ShapeDtypeStruct((nb, HIDDEN), jnp.float32),
    )(x, h, W_ih, W_hh, b_ih2d, b_hh2d)


def _topk_body(logp_ref, sc_ref, vals_ref, flats_ref):
    nb = logp_ref.shape[0]
    D = logp_ref.shape[1]
    cand = sc_ref[...] + logp_ref[...]
    row = jax.lax.broadcasted_iota(jnp.int32, (nb, D), 0)
    col = jax.lax.broadcasted_iota(jnp.int32, (nb, D), 1)
    iota_flat = row * D + col
    lane = jax.lax.broadcasted_iota(jnp.int32, (1, K), 1)
    big = jnp.int32(2 ** 31 - 1)
    neg = jnp.float32(-jnp.inf)

    def body(j, carry):
        work, vals, flats = carry
        m = jnp.max(work)
        flat = jnp.min(jnp.where(work == m, iota_flat, big))
        vals = jnp.where(lane == j, m, vals)
        flats = jnp.where(lane == j, flat, flats)
        work = jnp.where(iota_flat == flat, neg, work)
        return work, vals, flats

    vals0 = jnp.zeros((1, K), jnp.float32)
    flats0 = jnp.zeros((1, K), jnp.int32)
    _, vals, flats = jax.lax.fori_loop(0, K, body, (cand, vals0, flats0))
    vals_ref[...] = vals
    flats_ref[...] = flats


def _pallas_topk(logp, scores_col):
    nb = logp.shape[0]
    return pl.pallas_call(
        _topk_body,
        out_shape=(
            jax.ShapeDtypeStruct((1, K), jnp.float32),
            jax.ShapeDtypeStruct((1, K), jnp.int32),
        ),
    )(logp, scores_col)


def kernel(env, root_state, emb, W_ih, W_hh, b_ih, b_hh, W_out, b_out, beams, temp):
    B = env.shape[0]
    H = root_state.shape[1]
    D = W_out.shape[1]
    b_out2d = b_out.reshape(1, D)
    b_ih2d = b_ih.reshape(1, 3 * H)
    b_hh2d = b_hh.reshape(1, 3 * H)

    states = root_state
    scores = jnp.zeros((1, 1), dtype=jnp.float32)
    seqs = jnp.zeros((1, 0), dtype=jnp.int32)
    for _ in range(ENV_DEPTH):
        nb = states.shape[0]
        if PALLAS_LOGITS:
            logits = _pallas_logits(states, W_out, b_out2d)
        else:
            logits = states @ W_out + b_out
        logp_full = jax.nn.log_softmax(
            jnp.broadcast_to(logits[None], (B, nb, D)) / temp, axis=-1)
        logp = logp_full[0]
        k = min(64, nb * D)
        if PALLAS_TOPK:
            vals, flats = _pallas_topk(logp, scores)
            top_scores = vals[0]
            top_idx = flats[0]
        else:
            cand = (scores + logp).reshape(1, nb * D)
            ts, ti = jax.lax.top_k(cand, k)
            top_scores, top_idx = ts[0], ti[0]
        beam_idx = top_idx // D
        action = (top_idx % D).astype(jnp.int32)
        sel = states[beam_idx]
        seqs = jnp.concatenate([seqs[beam_idx], action[:, None]], axis=-1)
        x = emb[action]
        if PALLAS_GRU:
            states = _pallas_gru(x, sel, W_ih, W_hh, b_ih2d, b_hh2d)
        else:
            gi = x @ W_ih.T + b_ih
            gh = sel @ W_hh.T + b_hh
            i_r, i_z, i_n = jnp.split(gi, 3, axis=-1)
            h_r, h_z, h_n = jnp.split(gh, 3, axis=-1)
            r = jax.nn.sigmoid(i_r + h_r)
            z = jax.nn.sigmoid(i_z + h_z)
            n = jnp.tanh(i_n + r * h_n)
            states = (1.0 - z) * n + z * h
        scores = top_scores[:, None]

    out_seqs = jnp.broadcast_to(seqs[None], (B, K, ENV_DEPTH))
    out_scores = jnp.broadcast_to(top_scores[None], (B, K))
    return (out_seqs, out_scores)


# speedup vs baseline: 7.4439x; 1.5076x over previous
import jax
import jax.numpy as jnp
from jax.experimental import pallas as pl
from jax.experimental.pallas import tpu as pltpu

HIDDEN = 1024
DECISIONS = 1024
ENV_DEPTH = 8
BATCH = 16
K = 64


def _softmax_cand(logits, temp, scores_col):
    x = logits / temp
    m = jnp.max(x, axis=-1, keepdims=True)
    shifted = x - m
    lse = jnp.log(jnp.sum(jnp.exp(shifted), axis=-1, keepdims=True))
    return scores_col + (shifted - lse)


def _mono_body(root_ref, emb_ref, wih_ref, whh_ref, bih_ref, bhh_ref,
               wout_ref, bout_ref, t_ref,
               oseq_ref, osc_ref,
               st_a, st_b, x_buf, sel_buf, seq_a, seq_b):
    t = t_ref[0]
    D = DECISIONS
    H = HIDDEN
    dn = (((1,), (1,)), ((), ()))

    def depth_step(d, states_val, scores_col, st_src, seq_src, seq_dst):
        nb = states_val.shape[0]
        logits = jnp.dot(states_val, wout_ref[...],
                         preferred_element_type=jnp.float32) + bout_ref[...]
        cand = _softmax_cand(logits, t, scores_col)
        row = jax.lax.broadcasted_iota(jnp.int32, (nb, D), 0)
        col = jax.lax.broadcasted_iota(jnp.int32, (nb, D), 1)
        iota_flat = row * D + col
        lane = jax.lax.broadcasted_iota(jnp.int32, (1, K), 1)
        rowi = jax.lax.broadcasted_iota(jnp.int32, (K, 1), 0)
        big = jnp.int32(2 ** 31 - 1)
        neg = jnp.float32(-jnp.inf)

        def body(j, carry):
            work, vals_lane, vals_col = carry
            m = jnp.max(work)
            flat = jnp.min(jnp.where(work == m, iota_flat, big))
            bi = flat // D
            a = flat - bi * D
            vals_lane = jnp.where(lane == j, m, vals_lane)
            vals_col = jnp.where(rowi == j, m, vals_col)
            work = jnp.where(iota_flat == flat, neg, work)
            x_buf[pl.ds(j, 1), :] = emb_ref[pl.ds(a, 1), :]
            sel_buf[pl.ds(j, 1), :] = st_src[pl.ds(bi, 1), :]
            if d > 0:
                seq_dst[pl.ds(j, 1), 0:d] = seq_src[pl.ds(bi, 1), 0:d]
            seq_dst[pl.ds(j, 1), d:d + 1] = jnp.broadcast_to(a, (1, 1))
            return work, vals_lane, vals_col

        vals0 = jnp.zeros((1, K), jnp.float32)
        cols0 = jnp.zeros((K, 1), jnp.float32)
        _, vals_lane, vals_col = jax.lax.fori_loop(
            0, K, body, (cand, vals0, cols0))
        x = x_buf[...]
        h = sel_buf[...]
        gi = jax.lax.dot_general(x, wih_ref[...], dn,
                                 preferred_element_type=jnp.float32) + bih_ref[...]
        gh = jax.lax.dot_general(h, whh_ref[...], dn,
                                 preferred_element_type=jnp.float32) + bhh_ref[...]
        i_r, i_z, i_n = gi[:, 0:H], gi[:, H:2 * H], gi[:, 2 * H:3 * H]
        h_r, h_z, h_n = gh[:, 0:H], gh[:, H:2 * H], gh[:, 2 * H:3 * H]
        r = jax.nn.sigmoid(i_r + h_r)
        z = jax.nn.sigmoid(i_z + h_z)
        n = jnp.tanh(i_n + r * h_n)
        return (1.0 - z) * n + z * h, vals_lane, vals_col

    states = root_ref[...]
    scores_col = jnp.zeros((1, 1), jnp.float32)
    vals_lane = None
    for d in range(ENV_DEPTH):
        st_src = root_ref if d == 0 else (st_a if d % 2 == 1 else st_b)
        st_dst = st_b if d % 2 == 1 else st_a
        seq_src = seq_a if d % 2 == 1 else seq_b
        seq_dst = seq_b if d % 2 == 1 else seq_a
        if d == 0:
            st_dst, seq_dst, seq_src = st_a, seq_a, seq_b
        new_states, vals_lane, scores_col = depth_step(
            d, states, scores_col, st_src, seq_src, seq_dst)
        st_dst[...] = new_states
        states = new_states
        last_seq = seq_dst

    osc_ref[...] = jnp.broadcast_to(vals_lane, (BATCH, K))
    seq_final = last_seq[...]
    for b in range(BATCH):
        oseq_ref[b] = seq_final


def kernel(env, root_state, emb, W_ih, W_hh, b_ih, b_hh, W_out, b_out, beams, temp):
    H = root_state.shape[1]
    D = W_out.shape[1]
    tempf = jnp.asarray(temp, jnp.float32).reshape(1)
    out_seqs, out_scores = pl.pallas_call(
        _mono_body,
        in_specs=[
            pl.BlockSpec(memory_space=pltpu.VMEM),
            pl.BlockSpec(memory_space=pltpu.VMEM),
            pl.BlockSpec(memory_space=pltpu.VMEM),
            pl.BlockSpec(memory_space=pltpu.VMEM),
            pl.BlockSpec(memory_space=pltpu.VMEM),
            pl.BlockSpec(memory_space=pltpu.VMEM),
            pl.BlockSpec(memory_space=pltpu.VMEM),
            pl.BlockSpec(memory_space=pltpu.VMEM),
            pl.BlockSpec(memory_space=pltpu.SMEM),
        ],
        out_shape=(
            jax.ShapeDtypeStruct((BATCH, K, ENV_DEPTH), jnp.int32),
            jax.ShapeDtypeStruct((BATCH, K), jnp.float32),
        ),
        scratch_shapes=[
            pltpu.VMEM((K, HIDDEN), jnp.float32),
            pltpu.VMEM((K, HIDDEN), jnp.float32),
            pltpu.VMEM((K, HIDDEN), jnp.float32),
            pltpu.VMEM((K, HIDDEN), jnp.float32),
            pltpu.VMEM((K, ENV_DEPTH), jnp.int32),
            pltpu.VMEM((K, ENV_DEPTH), jnp.int32),
        ],
    )(root_state, emb, W_ih, W_hh,
      b_ih.reshape(1, 3 * H), b_hh.reshape(1, 3 * H),
      W_out, b_out.reshape(1, D), tempf)
    return (out_seqs, out_scores)


# speedup vs baseline: 7.8508x; 1.0547x over previous
import jax
import jax.numpy as jnp
from jax.experimental import pallas as pl
from jax.experimental.pallas import tpu as pltpu

HIDDEN = 1024
DECISIONS = 1024
ENV_DEPTH = 8
BATCH = 16
K = 64


def _softmax_cand(logits, temp, scores_col):
    x = logits / temp
    m = jnp.max(x, axis=-1, keepdims=True)
    shifted = x - m
    lse = jnp.log(jnp.sum(jnp.exp(shifted), axis=-1, keepdims=True))
    return scores_col + (shifted - lse)


def _mono_body(root_ref, emb_ref, wih_ref, whh_ref, bih_ref, bhh_ref,
               wout_ref, bout_ref, t_ref, oseq_ref, osc_ref):
    t = t_ref[0]
    D = DECISIONS
    H = HIDDEN
    dn = (((1,), (1,)), ((), ()))
    lane = jax.lax.broadcasted_iota(jnp.int32, (1, K), 1)
    rowi = jax.lax.broadcasted_iota(jnp.int32, (K, 1), 0)
    colD = jax.lax.broadcasted_iota(jnp.int32, (K, D), 1)
    colK = jax.lax.broadcasted_iota(jnp.int32, (K, K), 1)
    colS = jax.lax.broadcasted_iota(jnp.int32, (K, ENV_DEPTH), 1)
    big = jnp.int32(2 ** 31 - 1)
    neg = jnp.float32(-jnp.inf)

    def top64(cand):
        nb = cand.shape[0]
        row = jax.lax.broadcasted_iota(jnp.int32, (nb, D), 0)
        col = jax.lax.broadcasted_iota(jnp.int32, (nb, D), 1)
        iota_flat = row * D + col

        def body(j, carry):
            work, vals_lane, vals_col, a_col, bi_col = carry
            m = jnp.max(jnp.max(work, axis=1, keepdims=True),
                        axis=0, keepdims=True)
            flat = jnp.min(jnp.min(jnp.where(work == m, iota_flat, big),
                                   axis=1, keepdims=True),
                           axis=0, keepdims=True)
            bi = flat // D
            a = flat - bi * D
            sel = rowi == j
            vals_lane = jnp.where(lane == j, m, vals_lane)
            vals_col = jnp.where(sel, m, vals_col)
            a_col = jnp.where(sel, a, a_col)
            bi_col = jnp.where(sel, bi, bi_col)
            work = jnp.where(iota_flat == flat, neg, work)
            return work, vals_lane, vals_col, a_col, bi_col

        init = (cand,
                jnp.zeros((1, K), jnp.float32), jnp.zeros((K, 1), jnp.float32),
                jnp.zeros((K, 1), jnp.int32), jnp.zeros((K, 1), jnp.int32))
        _, vals_lane, vals_col, a_col, bi_col = jax.lax.fori_loop(
            0, K, body, init)
        return vals_lane, vals_col, a_col, bi_col

    states = root_ref[...]
    scores_col = jnp.zeros((1, 1), jnp.float32)
    seqs_f = jnp.zeros((K, ENV_DEPTH), jnp.float32)
    vals_lane = None
    for d in range(ENV_DEPTH):
        nb = states.shape[0]
        logits = jnp.dot(states, wout_ref[...],
                         preferred_element_type=jnp.float32) + bout_ref[...]
        cand = _softmax_cand(logits, t, scores_col)
        vals_lane, scores_col, a_col, bi_col = top64(cand)
        hp = jax.lax.Precision.HIGHEST
        onehot_a = (colD == a_col).astype(jnp.float32)
        x = jnp.dot(onehot_a, emb_ref[...], precision=hp,
                    preferred_element_type=jnp.float32)
        if d == 0:
            h = jnp.broadcast_to(states, (K, H))
            seqs_f = jnp.where(colS == d, a_col.astype(jnp.float32), seqs_f)
        else:
            onehot_b = (colK == bi_col).astype(jnp.float32)
            h = jnp.dot(onehot_b, states, precision=hp,
                        preferred_element_type=jnp.float32)
            seqs_f = jnp.dot(onehot_b, seqs_f, precision=hp,
                             preferred_element_type=jnp.float32)
            seqs_f = jnp.where(colS == d, a_col.astype(jnp.float32), seqs_f)
        gi = jax.lax.dot_general(x, wih_ref[...], dn,
                                 preferred_element_type=jnp.float32) + bih_ref[...]
        gh = jax.lax.dot_general(h, whh_ref[...], dn,
                                 preferred_element_type=jnp.float32) + bhh_ref[...]
        i_r, i_z, i_n = gi[:, 0:H], gi[:, H:2 * H], gi[:, 2 * H:3 * H]
        h_r, h_z, h_n = gh[:, 0:H], gh[:, H:2 * H], gh[:, 2 * H:3 * H]
        r = jax.nn.sigmoid(i_r + h_r)
        z = jax.nn.sigmoid(i_z + h_z)
        n = jnp.tanh(i_n + r * h_n)
        states = (1.0 - z) * n + z * h

    osc_ref[...] = jnp.broadcast_to(vals_lane, (BATCH, K))
    seqs_i = seqs_f.astype(jnp.int32)
    for b in range(BATCH):
        oseq_ref[b] = seqs_i


def kernel(env, root_state, emb, W_ih, W_hh, b_ih, b_hh, W_out, b_out, beams, temp):
    H = root_state.shape[1]
    D = W_out.shape[1]
    tempf = jnp.asarray(temp, jnp.float32).reshape(1)
    out_seqs, out_scores = pl.pallas_call(
        _mono_body,
        in_specs=[
            pl.BlockSpec(memory_space=pltpu.VMEM),
            pl.BlockSpec(memory_space=pltpu.VMEM),
            pl.BlockSpec(memory_space=pltpu.VMEM),
            pl.BlockSpec(memory_space=pltpu.VMEM),
            pl.BlockSpec(memory_space=pltpu.VMEM),
            pl.BlockSpec(memory_space=pltpu.VMEM),
            pl.BlockSpec(memory_space=pltpu.VMEM),
            pl.BlockSpec(memory_space=pltpu.VMEM),
            pl.BlockSpec(memory_space=pltpu.SMEM),
        ],
        out_shape=(
            jax.ShapeDtypeStruct((BATCH, K, ENV_DEPTH), jnp.int32),
            jax.ShapeDtypeStruct((BATCH, K), jnp.float32),
        ),
    )(root_state, emb, W_ih, W_hh,
      b_ih.reshape(1, 3 * H), b_hh.reshape(1, 3 * H),
      W_out, b_out.reshape(1, D), tempf)
    return (out_seqs, out_scores)
